# baseline (speedup 1.0000x reference)
import functools

import jax
import jax.numpy as jnp
from jax import lax
from jax.experimental import pallas as pl
from jax.experimental.pallas import tpu as pltpu
from jax.experimental.pallas import tpu_sc as plsc

VOCAB_SIZE = 1000
EMBED_DIM = 128
BATCH = 4096
SEQ = 200

NC = 2
NS = 16
NW = NC * NS

B = BATCH * SEQ
B_PER_W = B // NW
ROWS = 128
K = 2
GROUP = K * ROWS
NG = B_PER_W // GROUP
NSLOT = 2
NIDX = B_PER_W // ROWS


def _make_kernel():
    mesh = plsc.VectorSubcoreMesh(
        core_axis_name="c", subcore_axis_name="s",
        num_cores=NC, num_subcores=NS)

    @functools.partial(
        pl.kernel,
        mesh=mesh,
        out_type=jax.ShapeDtypeStruct((B, EMBED_DIM), jnp.float32),
        scratch_types=[
            pltpu.VMEM((NIDX, ROWS), jnp.int32),
            pltpu.VMEM((NSLOT, GROUP, EMBED_DIM), jnp.float32),
            pltpu.VMEM_SHARED((VOCAB_SIZE, EMBED_DIM), jnp.float32),
            [pltpu.SemaphoreType.DMA] * NSLOT,
            pltpu.SemaphoreType.DMA,
        ],
    )
    def emb_kernel(x_hbm, w_hbm, out_hbm, idx_v, rows_v, w_sh, gsems, osem):
        sid = lax.axis_index("s")
        wid = sid * NC + lax.axis_index("c")
        base = wid * B_PER_W

        @pl.when(sid == 0)
        def _stage_table():
            pltpu.sync_copy(w_hbm, w_sh)

        pltpu.sync_copy(x_hbm.at[wid], idx_v)
        plsc.subcore_barrier()

        def fire_gathers(g, slot):
            for t in range(K):
                pltpu.async_copy(
                    w_sh.at[idx_v.at[g * K + t]],
                    rows_v.at[slot, pl.ds(t * ROWS, ROWS)],
                    gsems[slot])

        def consume(g, slot, fire_g=None):
            for t in range(K):
                pltpu.make_async_copy(
                    w_sh.at[idx_v.at[g * K + t]],
                    rows_v.at[slot, pl.ds(t * ROWS, ROWS)],
                    gsems[slot]).wait()
            cp = pltpu.async_copy(
                rows_v.at[slot], out_hbm.at[pl.ds(base + g * GROUP, GROUP)],
                osem)
            cp.wait()
            if fire_g is not None:
                fire_gathers(fire_g, (slot + NSLOT - 1) % NSLOT)

        for g in range(NSLOT - 1):
            fire_gathers(g, g)

        def body(m, carry):
            for u in range(NSLOT):
                g = m * NSLOT + u
                consume(g, u, fire_g=g + NSLOT - 1)
            return carry

        n_main = (NG - (NSLOT - 1)) // NSLOT
        lax.fori_loop(0, n_main, body, 0)

        for g in range(n_main * NSLOT, NG):
            nxt = g + NSLOT - 1
            consume(g, g % NSLOT, fire_g=nxt if nxt < NG else None)

    return emb_kernel


_emb = _make_kernel()


def kernel(x, weight):
    x3 = x.reshape(NW, NIDX, ROWS)
    out = _emb(x3, weight)
    return out.reshape(BATCH, SEQ, EMBED_DIM)


# speedup vs baseline: 1.5375x; 1.5375x over previous
import functools

import jax
import jax.numpy as jnp
from jax import lax
from jax.experimental import pallas as pl
from jax.experimental.pallas import tpu as pltpu
from jax.experimental.pallas import tpu_sc as plsc

VOCAB_SIZE = 1000
EMBED_DIM = 128
BATCH = 4096
SEQ = 200

NC = 2
NS = 16
NW = NC * NS

B = BATCH * SEQ
B_PER_W = B // NW
ROWS = 128
NCH = B_PER_W // ROWS
NSLOT = 5
F = 3


def _make_kernel():
    mesh = plsc.VectorSubcoreMesh(
        core_axis_name="c", subcore_axis_name="s",
        num_cores=NC, num_subcores=NS)

    @functools.partial(
        pl.kernel,
        mesh=mesh,
        out_type=jax.ShapeDtypeStruct((B, EMBED_DIM), jnp.float32),
        scratch_types=[
            pltpu.VMEM((NCH, ROWS), jnp.int32),
            pltpu.VMEM((NSLOT, ROWS, EMBED_DIM), jnp.float32),
            pltpu.VMEM_SHARED((VOCAB_SIZE, EMBED_DIM), jnp.float32),
            [pltpu.SemaphoreType.DMA] * NSLOT,
            [pltpu.SemaphoreType.DMA] * NSLOT,
        ],
    )
    def emb_kernel(x_hbm, w_hbm, out_hbm, idx_v, rows_v, w_sh, gsems, osems):
        sid = lax.axis_index("s")
        wid = sid * NC + lax.axis_index("c")
        base = wid * B_PER_W

        @pl.when(sid == 0)
        def _stage_table():
            pltpu.sync_copy(w_hbm, w_sh)

        pltpu.sync_copy(x_hbm.at[wid], idx_v)
        plsc.subcore_barrier()

        def fire_g(c, u):
            pltpu.async_copy(
                w_sh.at[idx_v.at[c]], rows_v.at[u], gsems[u])

        def drain_g(c, u):
            pltpu.make_async_copy(
                w_sh.at[idx_v.at[c]], rows_v.at[u], gsems[u]).wait()

        def fire_s(c, u):
            pltpu.async_copy(
                rows_v.at[u], out_hbm.at[pl.ds(base + c * ROWS, ROWS)],
                osems[u])

        def drain_s(c, u):
            pltpu.make_async_copy(
                rows_v.at[u], out_hbm.at[pl.ds(base + c * ROWS, ROWS)],
                osems[u]).wait()

        def emit(c, u, drain_old=True, fire_new=True):
            drain_g(c, u)
            fire_s(c, u)
            ju = (u + F) % NSLOT
            if drain_old:
                drain_s(c + F - NSLOT, ju)
            if fire_new:
                fire_g(c + F, ju)

        for j in range(F):
            fire_g(j, j)
        for c in range(NSLOT - F):
            emit(c, c % NSLOT, drain_old=False)
        for c in range(NSLOT - F, NSLOT):
            emit(c, c % NSLOT)

        top = ((NCH - F - 1) // NSLOT) * NSLOT

        def body(m, carry):
            for u in range(NSLOT):
                emit(m * NSLOT + u, u)
            return carry

        lax.fori_loop(1, top // NSLOT, body, 0)

        for c in range(top, NCH - F):
            emit(c, c % NSLOT)
        for c in range(NCH - F, NCH):
            emit(c, c % NSLOT, fire_new=False)
        for c in range(NCH - (NSLOT - F), NCH):
            drain_s(c, c % NSLOT)

    return emb_kernel


_emb = _make_kernel()


def kernel(x, weight):
    x3 = x.reshape(NW, NCH, ROWS)
    out = _emb(x3, weight)
    return out.reshape(BATCH, SEQ, EMBED_DIM)
